# baseline (speedup 1.0000x reference)
import functools

import jax
import jax.numpy as jnp
from jax import lax
from jax.experimental import pallas as pl
from jax.experimental.pallas import tpu as pltpu
from jax.experimental.pallas import tpu_sc as plsc

_NC = 2
_NS = 16
_NW = _NC * _NS
_L = 16


@functools.partial(jax.jit, static_argnames=("chunk",))
def _embedding_lookup(flat_tokens, token_table, position_embedding, chunk):
    total = flat_tokens.shape[0]
    V, D = token_table.shape
    T = position_embedding.shape[0]
    rows_per_w = total // _NW
    nchunks = rows_per_w // chunk

    mesh = plsc.VectorSubcoreMesh(core_axis_name="c", subcore_axis_name="s")

    @functools.partial(
        pl.kernel,
        mesh=mesh,
        compiler_params=pltpu.CompilerParams(use_tc_tiling_on_sc=False),
        out_type=jax.ShapeDtypeStruct((total, D), jnp.float32),
        scratch_types=[
            pltpu.VMEM((rows_per_w,), jnp.int32),
            pltpu.VMEM((chunk, D), jnp.float32),
            pltpu.VMEM((chunk, D), jnp.float32),
            pltpu.VMEM((chunk, D), jnp.float32),
            pltpu.SemaphoreType.DMA,
            pltpu.SemaphoreType.DMA,
            pltpu.SemaphoreType.DMA,
            pltpu.SemaphoreType.DMA,
        ],
    )
    def emb_kernel(tok_hbm, tab_hbm, pos_hbm, out_hbm,
                   idx_v, buf0, buf1, pos_rep, g_sem0, g_sem1, w_sem0, w_sem1):
        wid = lax.axis_index("s") * _NC + lax.axis_index("c")
        base = wid * rows_per_w
        pltpu.sync_copy(tok_hbm.at[pl.ds(base, rows_per_w)], idx_v)
        for k in range(chunk // T):
            pltpu.sync_copy(pos_hbm, pos_rep.at[pl.ds(k * T, T)])

        bufs = (buf0, buf1)
        g_sems = (g_sem0, g_sem1)
        w_sems = (w_sem0, w_sem1)
        gcp = [None, None]
        wcp = [None, None]

        def start_gather(g, slot):
            return pltpu.async_copy(
                tab_hbm.at[idx_v.at[pl.ds(g * chunk, chunk)]],
                bufs[slot], g_sems[slot])

        gcp[0] = start_gather(0, 0)

        for g in range(nchunks):
            cur = g % 2
            nxt = (g + 1) % 2
            if g + 1 < nchunks:
                if wcp[nxt] is not None:
                    wcp[nxt].wait()
                    wcp[nxt] = None
                gcp[nxt] = start_gather(g + 1, nxt)
            gcp[cur].wait()

            buf = bufs[cur]

            @plsc.parallel_loop(0, chunk, unroll=4)
            def _add_pos(r):
                for c in range(D // _L):
                    p = pos_rep[r, pl.ds(c * _L, _L)]
                    plsc.addupdate(buf.at[r, pl.ds(c * _L, _L)], p)

            wcp[cur] = pltpu.async_copy(
                bufs[cur], out_hbm.at[pl.ds(base + g * chunk, chunk)],
                w_sems[cur])

        for cur in range(2):
            if wcp[cur] is not None:
                wcp[cur].wait()

    return emb_kernel(flat_tokens, token_table, position_embedding)


def kernel(tokens, token_table, position_embedding):
    B, T = tokens.shape
    D = token_table.shape[1]
    flat_tokens = tokens.reshape(B * T).astype(jnp.int32)
    out = _embedding_lookup(flat_tokens, token_table, position_embedding,
                            chunk=400)
    return out.reshape(B, T, D)


# speedup vs baseline: 1.0099x; 1.0099x over previous
import functools

import jax
import jax.numpy as jnp
from jax import lax
from jax.experimental import pallas as pl
from jax.experimental.pallas import tpu as pltpu
from jax.experimental.pallas import tpu_sc as plsc

_NC = 2
_NS = 16
_NW = _NC * _NS
_L = 16
_BBLK = 128


@jax.jit
def _embedding_lookup(tokens_t, token_table, position_embedding):
    T, B = tokens_t.shape
    V, D = token_table.shape
    CB = D // 8
    NB = B // _BBLK
    NG = _BBLK // _L

    mesh = plsc.VectorSubcoreMesh(core_axis_name="c", subcore_axis_name="s")

    @functools.partial(
        pl.kernel,
        mesh=mesh,
        compiler_params=pltpu.CompilerParams(
            use_tc_tiling_on_sc=False, needs_layout_passes=False),
        out_type=jax.ShapeDtypeStruct((T, CB, NB, 8, _BBLK), jnp.float32),
        scratch_types=[
            pltpu.VMEM((T, _BBLK), jnp.int32),
            pltpu.VMEM((_BBLK, D), jnp.float32),
            pltpu.VMEM((_BBLK, D), jnp.float32),
            pltpu.VMEM((CB, 8, _BBLK), jnp.float32),
            pltpu.VMEM((CB, 8, _BBLK), jnp.float32),
            pltpu.VMEM((T, D), jnp.float32),
            pltpu.SemaphoreType.DMA,
            pltpu.SemaphoreType.DMA,
            pltpu.SemaphoreType.DMA,
            pltpu.SemaphoreType.DMA,
        ],
    )
    def emb_kernel(tok_hbm, tab_hbm, pos_hbm, out_hbm,
                   idx_v, gbuf0, gbuf1, obuf0, obuf1, pos_v,
                   g_sem0, g_sem1, w_sem0, w_sem1):
        w = lax.axis_index("s") * _NC + lax.axis_index("c")
        pltpu.sync_copy(tok_hbm.at[:, pl.ds(w * _BBLK, _BBLK)], idx_v)
        pltpu.sync_copy(pos_hbm, pos_v)

        lane = lax.iota(jnp.int32, _L)
        row_ids = [lane + bg * _L for bg in range(NG)]

        gbufs = (gbuf0, gbuf1)
        obufs = (obuf0, obuf1)
        g_sems = (g_sem0, g_sem1)
        w_sems = (w_sem0, w_sem1)
        gcp = [None, None]
        wcp = [None, None]

        def start_gather(t, slot):
            return pltpu.async_copy(
                tab_hbm.at[idx_v.at[t]], gbufs[slot], g_sems[slot])

        gcp[0] = start_gather(0, 0)

        for t in range(T):
            cur = t % 2
            nxt = (t + 1) % 2
            if t + 1 < T:
                gcp[nxt] = start_gather(t + 1, nxt)
            gcp[cur].wait()
            if wcp[cur] is not None:
                wcp[cur].wait()
                wcp[cur] = None

            gbuf = gbufs[cur]
            obuf = obufs[cur]
            t_splat = jnp.full((_L,), t, dtype=jnp.int32)

            @plsc.parallel_loop(0, D, unroll=2)
            def _transpose_c(c):
                c_splat = jnp.full((_L,), 0, dtype=jnp.int32) + c
                p = plsc.load_gather(pos_v, [t_splat, c_splat])
                cb = lax.shift_right_logical(c, 3)
                ci = lax.bitwise_and(c, 7)
                for bg in range(NG):
                    v = plsc.load_gather(gbuf, [row_ids[bg], c_splat])
                    obuf[cb, ci, pl.ds(bg * _L, _L)] = v + p

            wcp[cur] = pltpu.async_copy(
                obuf, out_hbm.at[t, :, w, :, :], w_sems[cur])

        for cur in range(2):
            if wcp[cur] is not None:
                wcp[cur].wait()

    return emb_kernel(tokens_t, token_table, position_embedding)


def kernel(tokens, token_table, position_embedding):
    B, T = tokens.shape
    D = token_table.shape[1]
    tokens_t = tokens.T.astype(jnp.int32)
    out5 = _embedding_lookup(tokens_t, token_table, position_embedding)
    return out5.transpose(2, 4, 0, 1, 3).reshape(B, T, D)


# speedup vs baseline: 1.0608x; 1.0504x over previous
import functools

import jax
import jax.numpy as jnp
from jax import lax
from jax.experimental import pallas as pl
from jax.experimental.pallas import tpu as pltpu
from jax.experimental.pallas import tpu_sc as plsc

_NC = 2
_NS = 16
_NW = _NC * _NS
_L = 16
_BBLK = 128


@jax.jit
def _embedding_lookup(tokens_t, token_table, position_embedding):
    T, B = tokens_t.shape
    V, D = token_table.shape
    CB = D // 8
    NB = B // _BBLK
    NG = _BBLK // _L
    KP = T // 2

    mesh = plsc.VectorSubcoreMesh(core_axis_name="c", subcore_axis_name="s")

    @functools.partial(
        pl.kernel,
        mesh=mesh,
        compiler_params=pltpu.CompilerParams(
            use_tc_tiling_on_sc=False, needs_layout_passes=False),
        out_type=jax.ShapeDtypeStruct((T, CB, NB, 8, _BBLK), jnp.float32),
        scratch_types=[
            pltpu.VMEM((T, _BBLK), jnp.int32),
            pltpu.VMEM((_BBLK, D), jnp.float32),
            pltpu.VMEM((_BBLK, D), jnp.float32),
            pltpu.VMEM((CB, 8, _BBLK), jnp.float32),
            pltpu.VMEM((CB, 8, _BBLK), jnp.float32),
            pltpu.VMEM((T, D), jnp.float32),
            pltpu.SemaphoreType.DMA,
            pltpu.SemaphoreType.DMA,
            pltpu.SemaphoreType.DMA,
            pltpu.SemaphoreType.DMA,
        ],
    )
    def emb_kernel(tok_hbm, tab_hbm, pos_hbm, out_hbm,
                   idx_v, gbuf0, gbuf1, obuf0, obuf1, pos_v,
                   g_sem0, g_sem1, w_sem0, w_sem1):
        w = lax.axis_index("s") * _NC + lax.axis_index("c")
        pltpu.sync_copy(tok_hbm.at[:, pl.ds(w * _BBLK, _BBLK)], idx_v)
        pltpu.sync_copy(pos_hbm, pos_v)

        lane = lax.iota(jnp.int32, _L)
        row_ids = [lane + bg * _L for bg in range(NG)]

        def start_gather(t, gbuf, gsem):
            pltpu.async_copy(tab_hbm.at[idx_v.at[t]], gbuf, gsem)

        def wait_gather(t, gbuf, gsem):
            pltpu.make_async_copy(tab_hbm.at[idx_v.at[t]], gbuf, gsem).wait()

        def start_write(t, obuf, wsem):
            pltpu.async_copy(obuf, out_hbm.at[t, :, w, :, :], wsem)

        def wait_write(t, obuf, wsem):
            pltpu.make_async_copy(obuf, out_hbm.at[t, :, w, :, :], wsem).wait()

        def transpose_add(t, gbuf, obuf):
            t_splat = jnp.full((_L,), 0, dtype=jnp.int32) + t

            @plsc.parallel_loop(0, D, unroll=4)
            def _transpose_c(c):
                c_splat = jnp.full((_L,), 0, dtype=jnp.int32) + c
                p = plsc.load_gather(pos_v, [t_splat, c_splat])
                cb = lax.shift_right_logical(c, 3)
                ci = lax.bitwise_and(c, 7)
                for bg in range(NG):
                    v = plsc.load_gather(gbuf, [row_ids[bg], c_splat])
                    obuf[cb, ci, pl.ds(bg * _L, _L)] = v + p

        start_gather(0, gbuf0, g_sem0)
        start_gather(1, gbuf1, g_sem1)

        def pair_body(k, carry):
            t0 = 2 * k
            t1 = t0 + 1

            wait_gather(t0, gbuf0, g_sem0)

            @pl.when(k > 0)
            def _():
                wait_write(t0 - 2, obuf0, w_sem0)

            transpose_add(t0, gbuf0, obuf0)
            start_write(t0, obuf0, w_sem0)

            @pl.when(k < KP - 1)
            def _():
                start_gather(t0 + 2, gbuf0, g_sem0)

            wait_gather(t1, gbuf1, g_sem1)

            @pl.when(k > 0)
            def _():
                wait_write(t1 - 2, obuf1, w_sem1)

            transpose_add(t1, gbuf1, obuf1)
            start_write(t1, obuf1, w_sem1)

            @pl.when(k < KP - 1)
            def _():
                start_gather(t1 + 2, gbuf1, g_sem1)

            return carry

        lax.fori_loop(0, KP, pair_body, 0)

        wait_write(T - 2, obuf0, w_sem0)
        wait_write(T - 1, obuf1, w_sem1)

    return emb_kernel(tokens_t, token_table, position_embedding)


def kernel(tokens, token_table, position_embedding):
    B, T = tokens.shape
    D = token_table.shape[1]
    tokens_t = tokens.T.astype(jnp.int32)
    out5 = _embedding_lookup(tokens_t, token_table, position_embedding)
    return out5.transpose(2, 4, 0, 1, 3).reshape(B, T, D)
